# baseline (speedup 1.0000x reference)
import functools

import jax
import jax.numpy as jnp
from jax import lax
from jax.experimental import pallas as pl
from jax.experimental.pallas import tpu as pltpu
from jax.experimental.pallas import tpu_sc as plsc

N = 10000
NP = 10240
E = 320000
F1 = 128
F2 = 16
NC = 2
NS = 16
NW = NC * NS
K = 128
KG = 1
GSZ = KG * K
EPW = 10240
EP = NW * EPW
NG = EPW // GSZ
DKG = 4
DGSZ = DKG * K
NGD = EPW // DGSZ
RPT = NP // NS

_MESH = plsc.VectorSubcoreMesh(
    core_axis_name="c", subcore_axis_name="s", num_cores=NC, num_subcores=NS)


@functools.partial(
    pl.kernel,
    out_type=jax.ShapeDtypeStruct((NC, NP), jnp.float32),
    mesh=_MESH,
    scratch_types=[
        pltpu.VMEM((3 * DGSZ,), jnp.int32),
        pltpu.VMEM((K,), jnp.float32),
        pltpu.SemaphoreType.DMA,
        pltpu.SemaphoreType.DMA,
        pltpu.VMEM_SHARED((NP,), jnp.float32),
    ],
)
def _deg_sc(dst_ref, zeros_ref, ones_ref, out_ref, dbuf, ones_v, isem, ssem,
            acc):
    c = lax.axis_index("c")
    s = lax.axis_index("s")
    wid = c * NS + s
    pltpu.sync_copy(ones_ref, ones_v)
    pltpu.sync_copy(zeros_ref.at[pl.ds(s * RPT, RPT)],
                    acc.at[pl.ds(s * RPT, RPT)])
    plsc.subcore_barrier()

    def idx_issue(g):
        t = pl.multiple_of(lax.rem(g, 3) * DGSZ, DGSZ)
        base = pl.multiple_of(wid * EPW + g * DGSZ, DGSZ)
        pltpu.async_copy(dst_ref.at[pl.ds(base, DGSZ)],
                         dbuf.at[pl.ds(t, DGSZ)], isem)

    def idx_wait(g):
        t = pl.multiple_of(lax.rem(g, 3) * DGSZ, DGSZ)
        base = pl.multiple_of(wid * EPW + g * DGSZ, DGSZ)
        pltpu.make_async_copy(dst_ref.at[pl.ds(base, DGSZ)],
                              dbuf.at[pl.ds(t, DGSZ)], isem).wait()

    def scatter(g, issue):
        t = pl.multiple_of(lax.rem(g, 3) * DGSZ, DGSZ)
        for b in range(DKG):
            if issue:
                pltpu.async_copy(ones_v,
                                 acc.at[dbuf.at[pl.ds(t + b * K, K)]], ssem,
                                 add=True)
            else:
                pltpu.make_async_copy(ones_v,
                                      acc.at[dbuf.at[pl.ds(t + b * K, K)]],
                                      ssem).wait()

    idx_issue(0)
    idx_issue(1)
    idx_wait(0)

    def body(g, carry):
        @pl.when(g > 0)
        def _():
            scatter(g - 1, issue=False)

        @pl.when(g + 1 < NGD)
        def _():
            idx_wait(g + 1)

        @pl.when(g + 2 < NGD)
        def _():
            idx_issue(g + 2)

        scatter(g, issue=True)
        return carry

    lax.fori_loop(0, NGD, body, 0)
    scatter(NGD - 1, issue=False)
    plsc.subcore_barrier()
    pltpu.sync_copy(acc.at[pl.ds(s * RPT, RPT)],
                    out_ref.at[c, pl.ds(s * RPT, RPT)])


@functools.partial(
    pl.kernel,
    out_type=jax.ShapeDtypeStruct((NC, NP, F1), jnp.float32),
    mesh=_MESH,
    scratch_types=[
        pltpu.VMEM((3 * GSZ,), jnp.int32),
        pltpu.VMEM((3 * GSZ,), jnp.int32),
        pltpu.VMEM((2, KG, K, F1), jnp.float32),
        pltpu.SemaphoreType.DMA,
        pltpu.SemaphoreType.DMA,
        pltpu.SemaphoreType.DMA,
        pltpu.VMEM_SHARED((NP, F1), jnp.float32),
    ],
)
def _row_scatter(hp_ref, src_ref, dst_ref, zeros_ref, out_ref,
                 sbuf, dbuf, rows, isem, gsem, ssem, acc):
    c = lax.axis_index("c")
    s = lax.axis_index("s")
    wid = c * NS + s
    @pl.when(c == 0)
    def _():
        pltpu.sync_copy(hp_ref.at[pl.ds(s * RPT, RPT)],
                        acc.at[pl.ds(s * RPT, RPT)])

    @pl.when(c == 1)
    def _():
        pltpu.sync_copy(zeros_ref, acc.at[pl.ds(s * RPT, RPT)])

    plsc.subcore_barrier()

    def idx_issue(g):
        t = pl.multiple_of(lax.rem(g, 3) * GSZ, GSZ)
        base = pl.multiple_of(wid * EPW + g * GSZ, GSZ)
        pltpu.async_copy(src_ref.at[pl.ds(base, GSZ)],
                         sbuf.at[pl.ds(t, GSZ)], isem)
        pltpu.async_copy(dst_ref.at[pl.ds(base, GSZ)],
                         dbuf.at[pl.ds(t, GSZ)], isem)

    def idx_wait(g):
        t = pl.multiple_of(lax.rem(g, 3) * GSZ, GSZ)
        base = pl.multiple_of(wid * EPW + g * GSZ, GSZ)
        pltpu.make_async_copy(src_ref.at[pl.ds(base, GSZ)],
                              sbuf.at[pl.ds(t, GSZ)], isem).wait()
        pltpu.make_async_copy(dst_ref.at[pl.ds(base, GSZ)],
                              dbuf.at[pl.ds(t, GSZ)], isem).wait()

    def gather_issue(g):
        t = pl.multiple_of(lax.rem(g, 3) * GSZ, GSZ)
        r = lax.rem(g, 2)
        for b in range(KG):
            pltpu.async_copy(hp_ref.at[sbuf.at[pl.ds(t + b * K, K)]],
                             rows.at[r, b], gsem)

    def gather_wait(g):
        t = pl.multiple_of(lax.rem(g, 3) * GSZ, GSZ)
        r = lax.rem(g, 2)
        for b in range(KG):
            pltpu.make_async_copy(hp_ref.at[sbuf.at[pl.ds(t + b * K, K)]],
                                  rows.at[r, b], gsem).wait()

    def scatter_issue(g):
        t = pl.multiple_of(lax.rem(g, 3) * GSZ, GSZ)
        r = lax.rem(g, 2)
        for b in range(KG):
            pltpu.async_copy(rows.at[r, b],
                             acc.at[dbuf.at[pl.ds(t + b * K, K)]], ssem,
                             add=True)

    def scatter_wait(g):
        t = pl.multiple_of(lax.rem(g, 3) * GSZ, GSZ)
        r = lax.rem(g, 2)
        for b in range(KG):
            pltpu.make_async_copy(rows.at[r, b],
                                  acc.at[dbuf.at[pl.ds(t + b * K, K)]],
                                  ssem).wait()

    idx_issue(0)
    idx_issue(1)
    idx_wait(0)
    gather_issue(0)

    def body(g, carry):
        @pl.when(g > 0)
        def _():
            scatter_wait(g - 1)

        @pl.when(g + 1 < NG)
        def _():
            idx_wait(g + 1)
            gather_issue(g + 1)

        gather_wait(g)

        @pl.when(g + 2 < NG)
        def _():
            idx_issue(g + 2)

        scatter_issue(g)
        return carry

    lax.fori_loop(0, NG, body, 0)
    scatter_wait(NG - 1)
    plsc.subcore_barrier()
    pltpu.sync_copy(acc.at[pl.ds(s * RPT, RPT)],
                    out_ref.at[c, pl.ds(s * RPT, RPT)])


_RB = 2048


def _stage1_body(deg_ref, x_ref, w1_ref, dinv_ref, h1p_ref):
    deg = deg_ref[0, :] + deg_ref[1, :] + 1.0
    dinv = lax.rsqrt(deg)
    dinv_ref[...] = dinv[:, None]
    h = jnp.dot(x_ref[...], w1_ref[...], preferred_element_type=jnp.float32)
    h1p_ref[...] = h * dinv[:, None]


def _stage1(deg2, x_pad, W1):
    return pl.pallas_call(
        _stage1_body,
        grid=(NP // _RB,),
        in_specs=[
            pl.BlockSpec((2, _RB), lambda i: (0, i)),
            pl.BlockSpec((_RB, F1), lambda i: (i, 0)),
            pl.BlockSpec((F1, F1), lambda i: (0, 0)),
        ],
        out_specs=[
            pl.BlockSpec((_RB, 1), lambda i: (i, 0)),
            pl.BlockSpec((_RB, F1), lambda i: (i, 0)),
        ],
        out_shape=[
            jax.ShapeDtypeStruct((NP, 1), jnp.float32),
            jax.ShapeDtypeStruct((NP, F1), jnp.float32),
        ],
    )(deg2, x_pad, W1)


def _stage2_body(s1_ref, dinv_ref, b1_ref, g2_ref):
    t = s1_ref[0] + s1_ref[1]
    dinv = dinv_ref[...]
    out1 = jnp.maximum(dinv * t + b1_ref[...][None, :], 0.0)
    g2_ref[...] = out1 * dinv


def _stage2(s1, dinv, b1):
    return pl.pallas_call(
        _stage2_body,
        grid=(NP // _RB,),
        in_specs=[
            pl.BlockSpec((2, _RB, F1), lambda i: (0, i, 0)),
            pl.BlockSpec((_RB, 1), lambda i: (i, 0)),
            pl.BlockSpec((F1,), lambda i: (0,)),
        ],
        out_specs=pl.BlockSpec((_RB, F1), lambda i: (i, 0)),
        out_shape=jax.ShapeDtypeStruct((NP, F1), jnp.float32),
    )(s1, dinv, b1)


_RB3 = 2000


def _stage3_body(s2_ref, dinv_ref, b2_ref, w2_ref, out_ref):
    t = s2_ref[0] + s2_ref[1]
    h2 = jnp.dot(t, w2_ref[...], preferred_element_type=jnp.float32)
    o = dinv_ref[...] * h2 + b2_ref[...][None, :]
    m = jnp.max(o, axis=1, keepdims=True)
    lse = m + jnp.log(jnp.sum(jnp.exp(o - m), axis=1, keepdims=True))
    out_ref[...] = o - lse


def _stage3(s2, dinv, b2, W2):
    return pl.pallas_call(
        _stage3_body,
        grid=(N // _RB3,),
        in_specs=[
            pl.BlockSpec((2, _RB3, F1), lambda i: (0, i, 0)),
            pl.BlockSpec((_RB3, 1), lambda i: (i, 0)),
            pl.BlockSpec((F2,), lambda i: (0,)),
            pl.BlockSpec((F1, F2), lambda i: (0, 0)),
        ],
        out_specs=pl.BlockSpec((_RB3, F2), lambda i: (i, 0)),
        out_shape=jax.ShapeDtypeStruct((N, F2), jnp.float32),
    )(s2, dinv, b2, W2)


def kernel(x, edge_index, W1, b1, W2, b2):
    ei = edge_index.astype(jnp.int32)
    pad = N + jnp.arange(EP - E, dtype=jnp.int32) % (NP - N)
    src = jnp.concatenate([ei[0], pad])
    dst = jnp.concatenate([ei[1], pad])
    x_pad = jnp.zeros((NP, F1), jnp.float32).at[:N, :].set(x)
    zc = jnp.zeros((NP,), jnp.float32)
    ones = jnp.ones((K,), jnp.float32)
    zs = jnp.zeros((RPT, F1), jnp.float32)

    deg2 = _deg_sc(dst, zc, ones)
    dinv, h1p = _stage1(deg2, x_pad, W1)
    s1 = _row_scatter(h1p, src, dst, zs)
    g2 = _stage2(s1, dinv, b1)
    s2 = _row_scatter(g2, src, dst, zs)
    return _stage3(s2, dinv, b2, W2)


# speedup vs baseline: 1.0166x; 1.0166x over previous
import functools

import jax
import jax.numpy as jnp
from jax import lax
from jax.experimental import pallas as pl
from jax.experimental.pallas import tpu as pltpu
from jax.experimental.pallas import tpu_sc as plsc

N = 10000
NP = 10240
E = 320000
F1 = 128
F2 = 16
NC = 2
NS = 16
NW = NC * NS
K = 128
KG = 1
GSZ = KG * K
EPW = 10240
EP = NW * EPW
NG = EPW // GSZ
DKG = 8
DGSZ = DKG * K
NGD = EPW // DGSZ
RPT = NP // NS

_MESH = plsc.VectorSubcoreMesh(
    core_axis_name="c", subcore_axis_name="s", num_cores=NC, num_subcores=NS)


@functools.partial(
    pl.kernel,
    out_type=jax.ShapeDtypeStruct((NC, NP), jnp.float32),
    mesh=_MESH,
    scratch_types=[
        pltpu.VMEM((3 * DGSZ,), jnp.int32),
        pltpu.VMEM((K,), jnp.float32),
        pltpu.SemaphoreType.DMA,
        pltpu.SemaphoreType.DMA,
        pltpu.VMEM_SHARED((NP,), jnp.float32),
    ],
)
def _deg_sc(dst_ref, zeros_ref, ones_ref, out_ref, dbuf, ones_v, isem, ssem,
            acc):
    c = lax.axis_index("c")
    s = lax.axis_index("s")
    wid = c * NS + s
    pltpu.sync_copy(ones_ref, ones_v)
    pltpu.sync_copy(zeros_ref.at[pl.ds(s * RPT, RPT)],
                    acc.at[pl.ds(s * RPT, RPT)])
    plsc.subcore_barrier()

    def idx_issue(g):
        t = pl.multiple_of(lax.rem(g, 3) * DGSZ, DGSZ)
        base = pl.multiple_of(wid * EPW + g * DGSZ, DGSZ)
        pltpu.async_copy(dst_ref.at[pl.ds(base, DGSZ)],
                         dbuf.at[pl.ds(t, DGSZ)], isem)

    def idx_wait(g):
        t = pl.multiple_of(lax.rem(g, 3) * DGSZ, DGSZ)
        base = pl.multiple_of(wid * EPW + g * DGSZ, DGSZ)
        pltpu.make_async_copy(dst_ref.at[pl.ds(base, DGSZ)],
                              dbuf.at[pl.ds(t, DGSZ)], isem).wait()

    def scatter(g, issue):
        t = pl.multiple_of(lax.rem(g, 3) * DGSZ, DGSZ)
        for b in range(DKG):
            if issue:
                pltpu.async_copy(ones_v,
                                 acc.at[dbuf.at[pl.ds(t + b * K, K)]], ssem,
                                 add=True)
            else:
                pltpu.make_async_copy(ones_v,
                                      acc.at[dbuf.at[pl.ds(t + b * K, K)]],
                                      ssem).wait()

    idx_issue(0)
    idx_issue(1)
    idx_wait(0)

    def body(g, carry):
        @pl.when(g > 0)
        def _():
            scatter(g - 1, issue=False)

        @pl.when(g + 1 < NGD)
        def _():
            idx_wait(g + 1)

        @pl.when(g + 2 < NGD)
        def _():
            idx_issue(g + 2)

        scatter(g, issue=True)
        return carry

    lax.fori_loop(0, NGD, body, 0)
    scatter(NGD - 1, issue=False)
    plsc.subcore_barrier()
    pltpu.sync_copy(acc.at[pl.ds(s * RPT, RPT)],
                    out_ref.at[c, pl.ds(s * RPT, RPT)])


@functools.partial(
    pl.kernel,
    out_type=jax.ShapeDtypeStruct((NC, NP, F1), jnp.float32),
    mesh=_MESH,
    scratch_types=[
        pltpu.VMEM((3 * GSZ,), jnp.int32),
        pltpu.VMEM((3 * GSZ,), jnp.int32),
        pltpu.VMEM((2, KG, K, F1), jnp.float32),
        pltpu.SemaphoreType.DMA,
        pltpu.SemaphoreType.DMA,
        pltpu.SemaphoreType.DMA,
        pltpu.VMEM_SHARED((NP, F1), jnp.float32),
    ],
)
def _row_scatter(hp_ref, src_ref, dst_ref, zeros_ref, out_ref,
                 sbuf, dbuf, rows, isem, gsem, ssem, acc):
    c = lax.axis_index("c")
    s = lax.axis_index("s")
    wid = c * NS + s
    @pl.when(c == 0)
    def _():
        pltpu.sync_copy(hp_ref.at[pl.ds(s * RPT, RPT)],
                        acc.at[pl.ds(s * RPT, RPT)])

    @pl.when(c == 1)
    def _():
        pltpu.sync_copy(zeros_ref, acc.at[pl.ds(s * RPT, RPT)])

    plsc.subcore_barrier()

    def idx_issue(g):
        t = pl.multiple_of(lax.rem(g, 3) * GSZ, GSZ)
        base = pl.multiple_of(wid * EPW + g * GSZ, GSZ)
        pltpu.async_copy(src_ref.at[pl.ds(base, GSZ)],
                         sbuf.at[pl.ds(t, GSZ)], isem)
        pltpu.async_copy(dst_ref.at[pl.ds(base, GSZ)],
                         dbuf.at[pl.ds(t, GSZ)], isem)

    def idx_wait(g):
        t = pl.multiple_of(lax.rem(g, 3) * GSZ, GSZ)
        base = pl.multiple_of(wid * EPW + g * GSZ, GSZ)
        pltpu.make_async_copy(src_ref.at[pl.ds(base, GSZ)],
                              sbuf.at[pl.ds(t, GSZ)], isem).wait()
        pltpu.make_async_copy(dst_ref.at[pl.ds(base, GSZ)],
                              dbuf.at[pl.ds(t, GSZ)], isem).wait()

    def gather_issue(g):
        t = pl.multiple_of(lax.rem(g, 3) * GSZ, GSZ)
        r = lax.rem(g, 2)
        for b in range(KG):
            pltpu.async_copy(hp_ref.at[sbuf.at[pl.ds(t + b * K, K)]],
                             rows.at[r, b], gsem)

    def gather_wait(g):
        t = pl.multiple_of(lax.rem(g, 3) * GSZ, GSZ)
        r = lax.rem(g, 2)
        for b in range(KG):
            pltpu.make_async_copy(hp_ref.at[sbuf.at[pl.ds(t + b * K, K)]],
                                  rows.at[r, b], gsem).wait()

    def scatter_issue(g):
        t = pl.multiple_of(lax.rem(g, 3) * GSZ, GSZ)
        r = lax.rem(g, 2)
        for b in range(KG):
            pltpu.async_copy(rows.at[r, b],
                             acc.at[dbuf.at[pl.ds(t + b * K, K)]], ssem,
                             add=True)

    def scatter_wait(g):
        t = pl.multiple_of(lax.rem(g, 3) * GSZ, GSZ)
        r = lax.rem(g, 2)
        for b in range(KG):
            pltpu.make_async_copy(rows.at[r, b],
                                  acc.at[dbuf.at[pl.ds(t + b * K, K)]],
                                  ssem).wait()

    idx_issue(0)
    idx_issue(1)
    idx_wait(0)
    gather_issue(0)

    def body(g, carry):
        @pl.when(g > 0)
        def _():
            scatter_wait(g - 1)

        @pl.when(g + 1 < NG)
        def _():
            idx_wait(g + 1)
            gather_issue(g + 1)

        gather_wait(g)

        @pl.when(g + 2 < NG)
        def _():
            idx_issue(g + 2)

        scatter_issue(g)
        return carry

    lax.fori_loop(0, NG, body, 0)
    scatter_wait(NG - 1)
    plsc.subcore_barrier()
    pltpu.sync_copy(acc.at[pl.ds(s * RPT, RPT)],
                    out_ref.at[c, pl.ds(s * RPT, RPT)])


_RB = 2048


def _stage1_body(deg_ref, x_ref, w1_ref, dinv_ref, h1p_ref):
    deg = deg_ref[0, :] + deg_ref[1, :] + 1.0
    dinv = lax.rsqrt(deg)
    dinv_ref[...] = dinv[:, None]
    h = jnp.dot(x_ref[...], w1_ref[...], preferred_element_type=jnp.float32)
    h1p_ref[...] = h * dinv[:, None]


def _stage1(deg2, x_pad, W1):
    return pl.pallas_call(
        _stage1_body,
        grid=(NP // _RB,),
        in_specs=[
            pl.BlockSpec((2, _RB), lambda i: (0, i)),
            pl.BlockSpec((_RB, F1), lambda i: (i, 0)),
            pl.BlockSpec((F1, F1), lambda i: (0, 0)),
        ],
        out_specs=[
            pl.BlockSpec((_RB, 1), lambda i: (i, 0)),
            pl.BlockSpec((_RB, F1), lambda i: (i, 0)),
        ],
        out_shape=[
            jax.ShapeDtypeStruct((NP, 1), jnp.float32),
            jax.ShapeDtypeStruct((NP, F1), jnp.float32),
        ],
    )(deg2, x_pad, W1)


def _stage2_body(s1_ref, dinv_ref, b1_ref, g2_ref):
    t = s1_ref[0] + s1_ref[1]
    dinv = dinv_ref[...]
    out1 = jnp.maximum(dinv * t + b1_ref[...][None, :], 0.0)
    g2_ref[...] = out1 * dinv


def _stage2(s1, dinv, b1):
    return pl.pallas_call(
        _stage2_body,
        grid=(NP // _RB,),
        in_specs=[
            pl.BlockSpec((2, _RB, F1), lambda i: (0, i, 0)),
            pl.BlockSpec((_RB, 1), lambda i: (i, 0)),
            pl.BlockSpec((F1,), lambda i: (0,)),
        ],
        out_specs=pl.BlockSpec((_RB, F1), lambda i: (i, 0)),
        out_shape=jax.ShapeDtypeStruct((NP, F1), jnp.float32),
    )(s1, dinv, b1)


_RB3 = 2000


def _stage3_body(s2_ref, dinv_ref, b2_ref, w2_ref, out_ref):
    t = s2_ref[0] + s2_ref[1]
    h2 = jnp.dot(t, w2_ref[...], preferred_element_type=jnp.float32)
    o = dinv_ref[...] * h2 + b2_ref[...][None, :]
    m = jnp.max(o, axis=1, keepdims=True)
    lse = m + jnp.log(jnp.sum(jnp.exp(o - m), axis=1, keepdims=True))
    out_ref[...] = o - lse


def _stage3(s2, dinv, b2, W2):
    return pl.pallas_call(
        _stage3_body,
        grid=(N // _RB3,),
        in_specs=[
            pl.BlockSpec((2, _RB3, F1), lambda i: (0, i, 0)),
            pl.BlockSpec((_RB3, 1), lambda i: (i, 0)),
            pl.BlockSpec((F2,), lambda i: (0,)),
            pl.BlockSpec((F1, F2), lambda i: (0, 0)),
        ],
        out_specs=pl.BlockSpec((_RB3, F2), lambda i: (i, 0)),
        out_shape=jax.ShapeDtypeStruct((N, F2), jnp.float32),
    )(s2, dinv, b2, W2)


def kernel(x, edge_index, W1, b1, W2, b2):
    ei = edge_index.astype(jnp.int32)
    pad = N + jnp.arange(EP - E, dtype=jnp.int32) % (NP - N)
    src = jnp.concatenate([ei[0], pad])
    dst = jnp.concatenate([ei[1], pad])
    x_pad = jnp.zeros((NP, F1), jnp.float32).at[:N, :].set(x)
    zc = jnp.zeros((NP,), jnp.float32)
    ones = jnp.ones((K,), jnp.float32)
    zs = jnp.zeros((RPT, F1), jnp.float32)

    deg2 = _deg_sc(dst, zc, ones)
    dinv, h1p = _stage1(deg2, x_pad, W1)
    s1 = _row_scatter(h1p, src, dst, zs)
    g2 = _stage2(s1, dinv, b1)
    s2 = _row_scatter(g2, src, dst, zs)
    return _stage3(s2, dinv, b2, W2)


# speedup vs baseline: 1.0239x; 1.0072x over previous
import functools

import jax
import jax.numpy as jnp
from jax import lax
from jax.experimental import pallas as pl
from jax.experimental.pallas import tpu as pltpu
from jax.experimental.pallas import tpu_sc as plsc

N = 10000
NP = 10240
E = 320000
F1 = 128
F2 = 16
NC = 2
NS = 16
NW = NC * NS
K = 128
KG = 1
GSZ = KG * K
EPW = 10240
EP = NW * EPW
NG = EPW // GSZ
DKG = 16
DGSZ = DKG * K
NGD = EPW // DGSZ
RPT = NP // NS

_MESH = plsc.VectorSubcoreMesh(
    core_axis_name="c", subcore_axis_name="s", num_cores=NC, num_subcores=NS)


@functools.partial(
    pl.kernel,
    out_type=jax.ShapeDtypeStruct((NC, NP), jnp.float32),
    mesh=_MESH,
    scratch_types=[
        pltpu.VMEM((3 * DGSZ,), jnp.int32),
        pltpu.VMEM((K,), jnp.float32),
        pltpu.SemaphoreType.DMA,
        pltpu.SemaphoreType.DMA,
        pltpu.VMEM_SHARED((NP,), jnp.float32),
    ],
)
def _deg_sc(dst_ref, zeros_ref, ones_ref, out_ref, dbuf, ones_v, isem, ssem,
            acc):
    c = lax.axis_index("c")
    s = lax.axis_index("s")
    wid = c * NS + s
    pltpu.sync_copy(ones_ref, ones_v)
    pltpu.sync_copy(zeros_ref.at[pl.ds(s * RPT, RPT)],
                    acc.at[pl.ds(s * RPT, RPT)])
    plsc.subcore_barrier()

    def idx_issue(g):
        t = pl.multiple_of(lax.rem(g, 3) * DGSZ, DGSZ)
        base = pl.multiple_of(wid * EPW + g * DGSZ, DGSZ)
        pltpu.async_copy(dst_ref.at[pl.ds(base, DGSZ)],
                         dbuf.at[pl.ds(t, DGSZ)], isem)

    def idx_wait(g):
        t = pl.multiple_of(lax.rem(g, 3) * DGSZ, DGSZ)
        base = pl.multiple_of(wid * EPW + g * DGSZ, DGSZ)
        pltpu.make_async_copy(dst_ref.at[pl.ds(base, DGSZ)],
                              dbuf.at[pl.ds(t, DGSZ)], isem).wait()

    def scatter(g, issue):
        t = pl.multiple_of(lax.rem(g, 3) * DGSZ, DGSZ)
        for b in range(DKG):
            if issue:
                pltpu.async_copy(ones_v,
                                 acc.at[dbuf.at[pl.ds(t + b * K, K)]], ssem,
                                 add=True)
            else:
                pltpu.make_async_copy(ones_v,
                                      acc.at[dbuf.at[pl.ds(t + b * K, K)]],
                                      ssem).wait()

    idx_issue(0)
    idx_issue(1)
    idx_wait(0)

    def body(g, carry):
        @pl.when(g > 0)
        def _():
            scatter(g - 1, issue=False)

        @pl.when(g + 1 < NGD)
        def _():
            idx_wait(g + 1)

        @pl.when(g + 2 < NGD)
        def _():
            idx_issue(g + 2)

        scatter(g, issue=True)
        return carry

    lax.fori_loop(0, NGD, body, 0)
    scatter(NGD - 1, issue=False)
    plsc.subcore_barrier()
    pltpu.sync_copy(acc.at[pl.ds(s * RPT, RPT)],
                    out_ref.at[c, pl.ds(s * RPT, RPT)])


@functools.partial(
    pl.kernel,
    out_type=jax.ShapeDtypeStruct((NC, NP, F1), jnp.float32),
    mesh=_MESH,
    scratch_types=[
        pltpu.VMEM((3 * GSZ,), jnp.int32),
        pltpu.VMEM((3 * GSZ,), jnp.int32),
        pltpu.VMEM((2, KG, K, F1), jnp.float32),
        pltpu.SemaphoreType.DMA,
        pltpu.SemaphoreType.DMA,
        pltpu.SemaphoreType.DMA,
        pltpu.VMEM_SHARED((NP, F1), jnp.float32),
    ],
)
def _row_scatter(hp_ref, src_ref, dst_ref, zeros_ref, out_ref,
                 sbuf, dbuf, rows, isem, gsem, ssem, acc):
    c = lax.axis_index("c")
    s = lax.axis_index("s")
    wid = c * NS + s
    @pl.when(c == 0)
    def _():
        pltpu.sync_copy(hp_ref.at[pl.ds(s * RPT, RPT)],
                        acc.at[pl.ds(s * RPT, RPT)])

    @pl.when(c == 1)
    def _():
        pltpu.sync_copy(zeros_ref, acc.at[pl.ds(s * RPT, RPT)])

    plsc.subcore_barrier()

    def idx_issue(g):
        t = pl.multiple_of(lax.rem(g, 3) * GSZ, GSZ)
        base = pl.multiple_of(wid * EPW + g * GSZ, GSZ)
        pltpu.async_copy(src_ref.at[pl.ds(base, GSZ)],
                         sbuf.at[pl.ds(t, GSZ)], isem)
        pltpu.async_copy(dst_ref.at[pl.ds(base, GSZ)],
                         dbuf.at[pl.ds(t, GSZ)], isem)

    def idx_wait(g):
        t = pl.multiple_of(lax.rem(g, 3) * GSZ, GSZ)
        base = pl.multiple_of(wid * EPW + g * GSZ, GSZ)
        pltpu.make_async_copy(src_ref.at[pl.ds(base, GSZ)],
                              sbuf.at[pl.ds(t, GSZ)], isem).wait()
        pltpu.make_async_copy(dst_ref.at[pl.ds(base, GSZ)],
                              dbuf.at[pl.ds(t, GSZ)], isem).wait()

    def gather_issue(g):
        t = pl.multiple_of(lax.rem(g, 3) * GSZ, GSZ)
        r = lax.rem(g, 2)
        for b in range(KG):
            pltpu.async_copy(hp_ref.at[sbuf.at[pl.ds(t + b * K, K)]],
                             rows.at[r, b], gsem)

    def gather_wait(g):
        t = pl.multiple_of(lax.rem(g, 3) * GSZ, GSZ)
        r = lax.rem(g, 2)
        for b in range(KG):
            pltpu.make_async_copy(hp_ref.at[sbuf.at[pl.ds(t + b * K, K)]],
                                  rows.at[r, b], gsem).wait()

    def scatter_issue(g):
        t = pl.multiple_of(lax.rem(g, 3) * GSZ, GSZ)
        r = lax.rem(g, 2)
        for b in range(KG):
            pltpu.async_copy(rows.at[r, b],
                             acc.at[dbuf.at[pl.ds(t + b * K, K)]], ssem,
                             add=True)

    def scatter_wait(g):
        t = pl.multiple_of(lax.rem(g, 3) * GSZ, GSZ)
        r = lax.rem(g, 2)
        for b in range(KG):
            pltpu.make_async_copy(rows.at[r, b],
                                  acc.at[dbuf.at[pl.ds(t + b * K, K)]],
                                  ssem).wait()

    idx_issue(0)
    idx_issue(1)
    idx_wait(0)
    gather_issue(0)

    def body(g, carry):
        @pl.when(g > 0)
        def _():
            scatter_wait(g - 1)

        @pl.when(g + 1 < NG)
        def _():
            idx_wait(g + 1)
            gather_issue(g + 1)

        gather_wait(g)

        @pl.when(g + 2 < NG)
        def _():
            idx_issue(g + 2)

        scatter_issue(g)
        return carry

    lax.fori_loop(0, NG, body, 0)
    scatter_wait(NG - 1)
    plsc.subcore_barrier()
    pltpu.sync_copy(acc.at[pl.ds(s * RPT, RPT)],
                    out_ref.at[c, pl.ds(s * RPT, RPT)])


_RB = 2048


def _stage1_body(deg_ref, x_ref, w1_ref, dinv_ref, h1p_ref):
    deg = deg_ref[0, :] + deg_ref[1, :] + 1.0
    dinv = lax.rsqrt(deg)
    dinv_ref[...] = dinv[:, None]
    h = jnp.dot(x_ref[...], w1_ref[...], preferred_element_type=jnp.float32)
    h1p_ref[...] = h * dinv[:, None]


def _stage1(deg2, x_pad, W1):
    return pl.pallas_call(
        _stage1_body,
        grid=(NP // _RB,),
        in_specs=[
            pl.BlockSpec((2, _RB), lambda i: (0, i)),
            pl.BlockSpec((_RB, F1), lambda i: (i, 0)),
            pl.BlockSpec((F1, F1), lambda i: (0, 0)),
        ],
        out_specs=[
            pl.BlockSpec((_RB, 1), lambda i: (i, 0)),
            pl.BlockSpec((_RB, F1), lambda i: (i, 0)),
        ],
        out_shape=[
            jax.ShapeDtypeStruct((NP, 1), jnp.float32),
            jax.ShapeDtypeStruct((NP, F1), jnp.float32),
        ],
    )(deg2, x_pad, W1)


def _stage2_body(s1_ref, dinv_ref, b1_ref, g2_ref):
    t = s1_ref[0] + s1_ref[1]
    dinv = dinv_ref[...]
    out1 = jnp.maximum(dinv * t + b1_ref[...][None, :], 0.0)
    g2_ref[...] = out1 * dinv


def _stage2(s1, dinv, b1):
    return pl.pallas_call(
        _stage2_body,
        grid=(NP // _RB,),
        in_specs=[
            pl.BlockSpec((2, _RB, F1), lambda i: (0, i, 0)),
            pl.BlockSpec((_RB, 1), lambda i: (i, 0)),
            pl.BlockSpec((F1,), lambda i: (0,)),
        ],
        out_specs=pl.BlockSpec((_RB, F1), lambda i: (i, 0)),
        out_shape=jax.ShapeDtypeStruct((NP, F1), jnp.float32),
    )(s1, dinv, b1)


_RB3 = 2000


def _stage3_body(s2_ref, dinv_ref, b2_ref, w2_ref, out_ref):
    t = s2_ref[0] + s2_ref[1]
    h2 = jnp.dot(t, w2_ref[...], preferred_element_type=jnp.float32)
    o = dinv_ref[...] * h2 + b2_ref[...][None, :]
    m = jnp.max(o, axis=1, keepdims=True)
    lse = m + jnp.log(jnp.sum(jnp.exp(o - m), axis=1, keepdims=True))
    out_ref[...] = o - lse


def _stage3(s2, dinv, b2, W2):
    return pl.pallas_call(
        _stage3_body,
        grid=(N // _RB3,),
        in_specs=[
            pl.BlockSpec((2, _RB3, F1), lambda i: (0, i, 0)),
            pl.BlockSpec((_RB3, 1), lambda i: (i, 0)),
            pl.BlockSpec((F2,), lambda i: (0,)),
            pl.BlockSpec((F1, F2), lambda i: (0, 0)),
        ],
        out_specs=pl.BlockSpec((_RB3, F2), lambda i: (i, 0)),
        out_shape=jax.ShapeDtypeStruct((N, F2), jnp.float32),
    )(s2, dinv, b2, W2)


def kernel(x, edge_index, W1, b1, W2, b2):
    ei = edge_index.astype(jnp.int32)
    pad = N + jnp.arange(EP - E, dtype=jnp.int32) % (NP - N)
    src = jnp.concatenate([ei[0], pad])
    dst = jnp.concatenate([ei[1], pad])
    x_pad = jnp.zeros((NP, F1), jnp.float32).at[:N, :].set(x)
    zc = jnp.zeros((NP,), jnp.float32)
    ones = jnp.ones((K,), jnp.float32)
    zs = jnp.zeros((RPT, F1), jnp.float32)

    deg2 = _deg_sc(dst, zc, ones)
    dinv, h1p = _stage1(deg2, x_pad, W1)
    s1 = _row_scatter(h1p, src, dst, zs)
    g2 = _stage2(s1, dinv, b1)
    s2 = _row_scatter(g2, src, dst, zs)
    return _stage3(s2, dinv, b2, W2)


# speedup vs baseline: 1.0302x; 1.0062x over previous
import functools

import jax
import jax.numpy as jnp
from jax import lax
from jax.experimental import pallas as pl
from jax.experimental.pallas import tpu as pltpu
from jax.experimental.pallas import tpu_sc as plsc

N = 10000
NP = 10240
E = 320000
F1 = 128
F2 = 16
NC = 2
NS = 16
NW = NC * NS
K = 128
KG = 1
GSZ = KG * K
EPW = 10240
EP = NW * EPW
NG = EPW // GSZ
DKG = 16
DGSZ = DKG * K
NGD = EPW // DGSZ
RPT = NP // NS

_MESH = plsc.VectorSubcoreMesh(
    core_axis_name="c", subcore_axis_name="s", num_cores=NC, num_subcores=NS)


@functools.partial(
    pl.kernel,
    out_type=jax.ShapeDtypeStruct((NC, NP), jnp.float32),
    mesh=_MESH,
    scratch_types=[
        pltpu.VMEM((3 * DGSZ,), jnp.int32),
        pltpu.VMEM((K,), jnp.float32),
        pltpu.SemaphoreType.DMA,
        pltpu.SemaphoreType.DMA,
        pltpu.VMEM_SHARED((NP,), jnp.float32),
    ],
)
def _deg_sc(dst_ref, zeros_ref, ones_ref, out_ref, dbuf, ones_v, isem, ssem,
            acc):
    c = lax.axis_index("c")
    s = lax.axis_index("s")
    wid = c * NS + s
    pltpu.sync_copy(ones_ref, ones_v)
    pltpu.sync_copy(zeros_ref.at[pl.ds(s * RPT, RPT)],
                    acc.at[pl.ds(s * RPT, RPT)])
    plsc.subcore_barrier()

    def idx_issue(g):
        t = pl.multiple_of(lax.rem(g, 3) * DGSZ, DGSZ)
        base = pl.multiple_of(wid * EPW + g * DGSZ, DGSZ)
        pltpu.async_copy(dst_ref.at[pl.ds(base, DGSZ)],
                         dbuf.at[pl.ds(t, DGSZ)], isem)

    def idx_wait(g):
        t = pl.multiple_of(lax.rem(g, 3) * DGSZ, DGSZ)
        base = pl.multiple_of(wid * EPW + g * DGSZ, DGSZ)
        pltpu.make_async_copy(dst_ref.at[pl.ds(base, DGSZ)],
                              dbuf.at[pl.ds(t, DGSZ)], isem).wait()

    def scatter(g, issue):
        t = pl.multiple_of(lax.rem(g, 3) * DGSZ, DGSZ)
        for b in range(DKG):
            if issue:
                pltpu.async_copy(ones_v,
                                 acc.at[dbuf.at[pl.ds(t + b * K, K)]], ssem,
                                 add=True)
            else:
                pltpu.make_async_copy(ones_v,
                                      acc.at[dbuf.at[pl.ds(t + b * K, K)]],
                                      ssem).wait()

    idx_issue(0)
    idx_issue(1)
    idx_wait(0)

    def body(g, carry):
        @pl.when(g > 0)
        def _():
            scatter(g - 1, issue=False)

        @pl.when(g + 1 < NGD)
        def _():
            idx_wait(g + 1)

        @pl.when(g + 2 < NGD)
        def _():
            idx_issue(g + 2)

        scatter(g, issue=True)
        return carry

    lax.fori_loop(0, NGD, body, 0)
    scatter(NGD - 1, issue=False)
    plsc.subcore_barrier()
    pltpu.sync_copy(acc.at[pl.ds(s * RPT, RPT)],
                    out_ref.at[c, pl.ds(s * RPT, RPT)])


@functools.partial(
    pl.kernel,
    out_type=jax.ShapeDtypeStruct((NC, NP, F1), jnp.float32),
    mesh=_MESH,
    scratch_types=[
        pltpu.VMEM((3 * GSZ,), jnp.int32),
        pltpu.VMEM((3 * GSZ,), jnp.int32),
        pltpu.VMEM((2, KG, K, F1), jnp.float32),
        pltpu.SemaphoreType.DMA,
        pltpu.SemaphoreType.DMA,
        pltpu.SemaphoreType.DMA,
        pltpu.VMEM_SHARED((NP, F1), jnp.float32),
    ],
)
def _row_scatter(hp_ref, src_ref, dst_ref, zeros_ref, out_ref,
                 sbuf, dbuf, rows, isem, gsem, ssem, acc):
    c = lax.axis_index("c")
    s = lax.axis_index("s")
    wid = c * NS + s

    def idx_issue(g):
        t = pl.multiple_of(lax.rem(g, 3) * GSZ, GSZ)
        base = pl.multiple_of(wid * EPW + g * GSZ, GSZ)
        pltpu.async_copy(src_ref.at[pl.ds(base, GSZ)],
                         sbuf.at[pl.ds(t, GSZ)], isem)
        pltpu.async_copy(dst_ref.at[pl.ds(base, GSZ)],
                         dbuf.at[pl.ds(t, GSZ)], isem)

    def idx_wait(g):
        t = pl.multiple_of(lax.rem(g, 3) * GSZ, GSZ)
        base = pl.multiple_of(wid * EPW + g * GSZ, GSZ)
        pltpu.make_async_copy(src_ref.at[pl.ds(base, GSZ)],
                              sbuf.at[pl.ds(t, GSZ)], isem).wait()
        pltpu.make_async_copy(dst_ref.at[pl.ds(base, GSZ)],
                              dbuf.at[pl.ds(t, GSZ)], isem).wait()

    def gather_issue(g):
        t = pl.multiple_of(lax.rem(g, 3) * GSZ, GSZ)
        r = lax.rem(g, 2)
        for b in range(KG):
            pltpu.async_copy(hp_ref.at[sbuf.at[pl.ds(t + b * K, K)]],
                             rows.at[r, b], gsem)

    def gather_wait(g):
        t = pl.multiple_of(lax.rem(g, 3) * GSZ, GSZ)
        r = lax.rem(g, 2)
        for b in range(KG):
            pltpu.make_async_copy(hp_ref.at[sbuf.at[pl.ds(t + b * K, K)]],
                                  rows.at[r, b], gsem).wait()

    def scatter_issue(g):
        t = pl.multiple_of(lax.rem(g, 3) * GSZ, GSZ)
        r = lax.rem(g, 2)
        for b in range(KG):
            pltpu.async_copy(rows.at[r, b],
                             acc.at[dbuf.at[pl.ds(t + b * K, K)]], ssem,
                             add=True)

    def scatter_wait(g):
        t = pl.multiple_of(lax.rem(g, 3) * GSZ, GSZ)
        r = lax.rem(g, 2)
        for b in range(KG):
            pltpu.make_async_copy(rows.at[r, b],
                                  acc.at[dbuf.at[pl.ds(t + b * K, K)]],
                                  ssem).wait()

    idx_issue(0)
    idx_issue(1)
    @pl.when(c == 0)
    def _():
        pltpu.sync_copy(hp_ref.at[pl.ds(s * RPT, RPT)],
                        acc.at[pl.ds(s * RPT, RPT)])

    @pl.when(c == 1)
    def _():
        pltpu.sync_copy(zeros_ref, acc.at[pl.ds(s * RPT, RPT)])

    idx_wait(0)
    gather_issue(0)
    plsc.subcore_barrier()

    def body(g, carry):
        @pl.when(g > 0)
        def _():
            scatter_wait(g - 1)

        @pl.when(g + 1 < NG)
        def _():
            idx_wait(g + 1)
            gather_issue(g + 1)

        gather_wait(g)

        @pl.when(g + 2 < NG)
        def _():
            idx_issue(g + 2)

        scatter_issue(g)
        return carry

    lax.fori_loop(0, NG, body, 0)
    scatter_wait(NG - 1)
    plsc.subcore_barrier()
    pltpu.sync_copy(acc.at[pl.ds(s * RPT, RPT)],
                    out_ref.at[c, pl.ds(s * RPT, RPT)])


_RB = 2048


def _stage1_body(deg_ref, x_ref, w1_ref, dinv_ref, h1p_ref):
    deg = deg_ref[0, :] + deg_ref[1, :] + 1.0
    dinv = lax.rsqrt(deg)
    dinv_ref[...] = dinv[:, None]
    h = jnp.dot(x_ref[...], w1_ref[...], preferred_element_type=jnp.float32)
    h1p_ref[...] = h * dinv[:, None]


def _stage1(deg2, x_pad, W1):
    return pl.pallas_call(
        _stage1_body,
        grid=(NP // _RB,),
        in_specs=[
            pl.BlockSpec((2, _RB), lambda i: (0, i)),
            pl.BlockSpec((_RB, F1), lambda i: (i, 0)),
            pl.BlockSpec((F1, F1), lambda i: (0, 0)),
        ],
        out_specs=[
            pl.BlockSpec((_RB, 1), lambda i: (i, 0)),
            pl.BlockSpec((_RB, F1), lambda i: (i, 0)),
        ],
        out_shape=[
            jax.ShapeDtypeStruct((NP, 1), jnp.float32),
            jax.ShapeDtypeStruct((NP, F1), jnp.float32),
        ],
    )(deg2, x_pad, W1)


def _stage2_body(s1_ref, dinv_ref, b1_ref, g2_ref):
    t = s1_ref[0] + s1_ref[1]
    dinv = dinv_ref[...]
    out1 = jnp.maximum(dinv * t + b1_ref[...][None, :], 0.0)
    g2_ref[...] = out1 * dinv


def _stage2(s1, dinv, b1):
    return pl.pallas_call(
        _stage2_body,
        grid=(NP // _RB,),
        in_specs=[
            pl.BlockSpec((2, _RB, F1), lambda i: (0, i, 0)),
            pl.BlockSpec((_RB, 1), lambda i: (i, 0)),
            pl.BlockSpec((F1,), lambda i: (0,)),
        ],
        out_specs=pl.BlockSpec((_RB, F1), lambda i: (i, 0)),
        out_shape=jax.ShapeDtypeStruct((NP, F1), jnp.float32),
    )(s1, dinv, b1)


_RB3 = 2000


def _stage3_body(s2_ref, dinv_ref, b2_ref, w2_ref, out_ref):
    t = s2_ref[0] + s2_ref[1]
    h2 = jnp.dot(t, w2_ref[...], preferred_element_type=jnp.float32)
    o = dinv_ref[...] * h2 + b2_ref[...][None, :]
    m = jnp.max(o, axis=1, keepdims=True)
    lse = m + jnp.log(jnp.sum(jnp.exp(o - m), axis=1, keepdims=True))
    out_ref[...] = o - lse


def _stage3(s2, dinv, b2, W2):
    return pl.pallas_call(
        _stage3_body,
        grid=(N // _RB3,),
        in_specs=[
            pl.BlockSpec((2, _RB3, F1), lambda i: (0, i, 0)),
            pl.BlockSpec((_RB3, 1), lambda i: (i, 0)),
            pl.BlockSpec((F2,), lambda i: (0,)),
            pl.BlockSpec((F1, F2), lambda i: (0, 0)),
        ],
        out_specs=pl.BlockSpec((_RB3, F2), lambda i: (i, 0)),
        out_shape=jax.ShapeDtypeStruct((N, F2), jnp.float32),
    )(s2, dinv, b2, W2)


def kernel(x, edge_index, W1, b1, W2, b2):
    ei = edge_index.astype(jnp.int32)
    pad = N + jnp.arange(EP - E, dtype=jnp.int32) % (NP - N)
    src = jnp.concatenate([ei[0], pad])
    dst = jnp.concatenate([ei[1], pad])
    x_pad = jnp.zeros((NP, F1), jnp.float32).at[:N, :].set(x)
    zc = jnp.zeros((NP,), jnp.float32)
    ones = jnp.ones((K,), jnp.float32)
    zs = jnp.zeros((RPT, F1), jnp.float32)

    deg2 = _deg_sc(dst, zc, ones)
    dinv, h1p = _stage1(deg2, x_pad, W1)
    s1 = _row_scatter(h1p, src, dst, zs)
    g2 = _stage2(s1, dinv, b1)
    s2 = _row_scatter(g2, src, dst, zs)
    return _stage3(s2, dinv, b2, W2)
